# Initial kernel scaffold; baseline (speedup 1.0000x reference)
import functools

import jax
import jax.numpy as jnp
from jax import lax
from jax.experimental import pallas as pl
from jax.experimental.pallas import tpu as pltpu
from jax.experimental.pallas import tpu_sc as plsc

B = 16384
D = 64
NIDX = 10

_NC = 2
_NS = 16
_NW = _NC * _NS
_RPW = B // _NW
_C = 128
_NCH = _RPW // _C

_mesh = plsc.VectorSubcoreMesh(core_axis_name="c", subcore_axis_name="s")


def _sc_gather_body(idx_hbm, sp_hbm, it_hbm, ab_hbm, mv_hbm, out_hbm,
                    idx_v, gbuf, sem):
    wid = lax.axis_index("s") * _NC + lax.axis_index("c")
    base = wid * _RPW

    def chunk(ci, carry):
        r = pl.multiple_of(base + ci * _C, _C)
        pltpu.sync_copy(idx_hbm.at[:, pl.ds(r, _C)], idx_v)
        tables = (sp_hbm, it_hbm, ab_hbm, ab_hbm, ab_hbm,
                  mv_hbm, mv_hbm, mv_hbm, mv_hbm, mv_hbm)
        cps = []
        for j, t in enumerate(tables):
            cps.append(pltpu.async_copy(t.at[idx_v.at[j]], gbuf.at[j], sem))
        for cp in cps:
            cp.wait()

        def abil_body(i, c):
            for k in range(4):
                s = pl.ds(k * 16, 16)
                gbuf[3, i, s] = (gbuf[3, i, s] + gbuf[4, i, s]
                                 + gbuf[5, i, s]) * (1.0 / 3.0)
            return c

        lax.fori_loop(0, _C, abil_body, 0)

        def mv_body(i, c):
            for k in range(4):
                s = pl.ds(k * 16, 16)
                gbuf[6, i, s] = (gbuf[6, i, s] + gbuf[7, i, s]
                                 + gbuf[8, i, s] + gbuf[9, i, s]) * 0.25
            return c

        lax.fori_loop(0, _C, mv_body, 0)

        for p, j in enumerate((0, 1, 2, 3, 6)):
            pltpu.sync_copy(gbuf.at[j], out_hbm.at[p, pl.ds(r, _C)])
        return carry

    lax.fori_loop(0, _NCH, chunk, 0)


_sc_gather = functools.partial(
    pl.kernel,
    out_type=jax.ShapeDtypeStruct((5, B, D), jnp.float32),
    mesh=_mesh,
    scratch_types=[
        pltpu.VMEM((NIDX, _C), jnp.int32),
        pltpu.VMEM((NIDX, _C, D), jnp.float32),
        pltpu.SemaphoreType.DMA,
    ],
)(_sc_gather_body)


_BC = 2048


def _tc_body(emb_ref, st_ref, w5_ref, wp_ref, b_ref, out_ref):
    dn = (((1,), (0,)), ((), ()))
    acc = lax.dot_general(st_ref[...], wp_ref[...], dn,
                          precision=lax.Precision.HIGHEST,
                          preferred_element_type=jnp.float32)
    for p in range(5):
        acc = acc + lax.dot_general(emb_ref[p], w5_ref[p], dn,
                                    precision=lax.Precision.HIGHEST,
                                    preferred_element_type=jnp.float32)
    out_ref[...] = jnp.maximum(acc + b_ref[...], 0.0)


_tc_matmul = pl.pallas_call(
    _tc_body,
    grid=(B // _BC,),
    in_specs=[
        pl.BlockSpec((5, _BC, D), lambda i: (0, i, 0)),
        pl.BlockSpec((_BC, NIDX + 128), lambda i: (i, 0)),
        pl.BlockSpec((5, D, D), lambda i: (0, 0, 0)),
        pl.BlockSpec((NIDX + 128, D), lambda i: (0, 0)),
        pl.BlockSpec((1, D), lambda i: (0, 0)),
    ],
    out_specs=pl.BlockSpec((_BC, D), lambda i: (i, 0)),
    out_shape=jax.ShapeDtypeStruct((B, D), jnp.float32),
)


def kernel(state, species_table, item_table, ability_table, move_table, W, b):
    idx = state[:, :NIDX].astype(jnp.int32).T
    emb = _sc_gather(idx, species_table, item_table, ability_table, move_table)
    W5 = W[:5 * D].reshape(5, D, D)
    Wp = jnp.concatenate([jnp.zeros((NIDX, D), W.dtype), W[5 * D:]], axis=0)
    b2 = b.reshape(1, D)
    return _tc_matmul(emb, state, W5, Wp, b2)


# speedup vs baseline: 2.3997x; 2.3997x over previous
import functools

import jax
import jax.numpy as jnp
from jax import lax
from jax.experimental import pallas as pl
from jax.experimental.pallas import tpu as pltpu
from jax.experimental.pallas import tpu_sc as plsc

B = 16384
D = 64
NIDX = 10

_NC = 2
_NS = 16
_NW = _NC * _NS
_RPW = B // _NW
_C = 128
_NCH = _RPW // _C

_mesh = plsc.VectorSubcoreMesh(core_axis_name="c", subcore_axis_name="s",
                               num_cores=_NC, num_subcores=_NS)


def _sc_gather_body(idx_hbm, sp_hbm, it_hbm, ab_hbm, mv_hbm, out_hbm,
                    idx_v, gbuf, sem):
    wid = lax.axis_index("s") * _NC + lax.axis_index("c")
    base = wid * _RPW

    def chunk(ci, carry):
        r = pl.multiple_of(base + ci * _C, _C)
        icps = [pltpu.async_copy(idx_hbm.at[j, pl.ds(r, _C)], idx_v.at[j],
                                 sem) for j in range(NIDX)]
        for cp in icps:
            cp.wait()
        tables = (sp_hbm, it_hbm, ab_hbm, ab_hbm, ab_hbm, ab_hbm,
                  mv_hbm, mv_hbm, mv_hbm, mv_hbm)
        cps = []
        for j, t in enumerate(tables):
            cps.append(pltpu.async_copy(t.at[idx_v.at[j]], gbuf.at[j], sem))
        for cp in cps:
            cp.wait()
        for j in range(NIDX):
            pltpu.sync_copy(gbuf.at[j], out_hbm.at[j, pl.ds(r, _C)])
        return carry

    lax.fori_loop(0, _NCH, chunk, 0)


_sc_gather = functools.partial(
    pl.kernel,
    out_type=jax.ShapeDtypeStruct((NIDX, B, D), jnp.float32),
    mesh=_mesh,
    compiler_params=pltpu.CompilerParams(use_tc_tiling_on_sc=False),
    scratch_types=[
        pltpu.VMEM((NIDX, _C), jnp.int32),
        pltpu.VMEM((NIDX, _C, D), jnp.float32),
        pltpu.SemaphoreType.DMA,
    ],
)(_sc_gather_body)


_BC = 2048


def _tc_body(emb_ref, st_ref, w10_ref, wp_ref, b_ref, out_ref):
    dn = (((1,), (0,)), ((), ()))
    acc = lax.dot_general(st_ref[...], wp_ref[...], dn,
                          precision=lax.Precision.HIGHEST,
                          preferred_element_type=jnp.float32)
    for j in range(NIDX):
        acc = acc + lax.dot_general(emb_ref[j], w10_ref[j], dn,
                                    precision=lax.Precision.HIGHEST,
                                    preferred_element_type=jnp.float32)
    out_ref[...] = jnp.maximum(acc + b_ref[...], 0.0)


_tc_matmul = pl.pallas_call(
    _tc_body,
    grid=(B // _BC,),
    in_specs=[
        pl.BlockSpec((NIDX, _BC, D), lambda i: (0, i, 0)),
        pl.BlockSpec((_BC, NIDX + 128), lambda i: (i, 0)),
        pl.BlockSpec((NIDX, D, D), lambda i: (0, 0, 0)),
        pl.BlockSpec((NIDX + 128, D), lambda i: (0, 0)),
        pl.BlockSpec((1, D), lambda i: (0, 0)),
    ],
    out_specs=pl.BlockSpec((_BC, D), lambda i: (i, 0)),
    out_shape=jax.ShapeDtypeStruct((B, D), jnp.float32),
)


def kernel(state, species_table, item_table, ability_table, move_table, W, b):
    idx = state[:, :NIDX].astype(jnp.int32).T
    emb = _sc_gather(idx, species_table, item_table, ability_table,
                     move_table)
    W10 = jnp.concatenate([
        W[:3 * D].reshape(3, D, D),
        jnp.broadcast_to(W[3 * D:4 * D] * (1.0 / 3.0), (3, D, D)),
        jnp.broadcast_to(W[4 * D:5 * D] * 0.25, (4, D, D)),
    ])
    Wp = jnp.concatenate([jnp.zeros((NIDX, D), W.dtype), W[5 * D:]], axis=0)
    b2 = b.reshape(1, D)
    return _tc_matmul(emb, state, W10, Wp, b2)


# speedup vs baseline: 3.0102x; 1.2544x over previous
import functools

import jax
import jax.numpy as jnp
from jax import lax
from jax.experimental import pallas as pl
from jax.experimental.pallas import tpu as pltpu
from jax.experimental.pallas import tpu_sc as plsc

B = 16384
D = 64
NIDX = 10

_NC = 2
_NS = 16
_NW = _NC * _NS
_RPW = B // _NW
_C = 64
_NCH = _RPW // _C

_mesh = plsc.VectorSubcoreMesh(core_axis_name="c", subcore_axis_name="s",
                               num_cores=_NC, num_subcores=_NS)


def _sc_gather_body(idx_hbm, sp_hbm, it_hbm, ab_hbm, mv_hbm, out_hbm,
                    idx_v, gbuf, gsem, wsem):
    wid = lax.axis_index("s") * _NC + lax.axis_index("c")
    base = wid * _RPW
    tables = (sp_hbm, it_hbm, ab_hbm, ab_hbm, ab_hbm, ab_hbm,
              mv_hbm, mv_hbm, mv_hbm, mv_hbm)

    pltpu.sync_copy(idx_hbm.at[:, pl.ds(base, _RPW)], idx_v)

    def fire_gathers(ci, par):
        cps = []
        for j, t in enumerate(tables):
            cps.append(pltpu.async_copy(
                t.at[idx_v.at[j, pl.ds(ci * _C, _C)]],
                gbuf.at[par, j], gsem.at[par]))
        return cps

    def fire_writes(ci, par):
        r = pl.multiple_of(base + ci * _C, _C)
        return [pltpu.async_copy(gbuf.at[par, j],
                                 out_hbm.at[j, pl.ds(r, _C)], wsem)
                for j in range(NIDX)]

    g_cps = {0: fire_gathers(0, 0)}
    w_cps = {}
    for ci in range(_NCH):
        par = ci % 2
        if ci + 1 < _NCH:
            nxt = (ci + 1) % 2
            if ci >= 1:
                for cp in w_cps.pop(ci - 1):
                    cp.wait()
            g_cps[ci + 1] = fire_gathers(ci + 1, nxt)
        for cp in g_cps.pop(ci):
            cp.wait()
        w_cps[ci] = fire_writes(ci, par)
    for cps in w_cps.values():
        for cp in cps:
            cp.wait()


_sc_gather = functools.partial(
    pl.kernel,
    out_type=jax.ShapeDtypeStruct((NIDX, B, D), jnp.float32),
    mesh=_mesh,
    compiler_params=pltpu.CompilerParams(use_tc_tiling_on_sc=False),
    scratch_types=[
        pltpu.VMEM((NIDX, _RPW), jnp.int32),
        pltpu.VMEM((2, NIDX, _C, D), jnp.float32),
        pltpu.SemaphoreType.DMA((2,)),
        pltpu.SemaphoreType.DMA,
    ],
)(_sc_gather_body)


_BC = 2048


def _tc_body(emb_ref, st_ref, w10_ref, wp_ref, b_ref, out_ref):
    dn = (((1,), (0,)), ((), ()))
    acc = lax.dot_general(st_ref[...], wp_ref[...], dn,
                          preferred_element_type=jnp.float32)
    for j in range(NIDX):
        acc = acc + lax.dot_general(emb_ref[j], w10_ref[j], dn,
                                    preferred_element_type=jnp.float32)
    out_ref[...] = jnp.maximum(acc + b_ref[...], 0.0)


_tc_matmul = pl.pallas_call(
    _tc_body,
    grid=(B // _BC,),
    in_specs=[
        pl.BlockSpec((NIDX, _BC, D), lambda i: (0, i, 0)),
        pl.BlockSpec((_BC, NIDX + 128), lambda i: (i, 0)),
        pl.BlockSpec((NIDX, D, D), lambda i: (0, 0, 0)),
        pl.BlockSpec((NIDX + 128, D), lambda i: (0, 0)),
        pl.BlockSpec((1, D), lambda i: (0, 0)),
    ],
    out_specs=pl.BlockSpec((_BC, D), lambda i: (i, 0)),
    out_shape=jax.ShapeDtypeStruct((B, D), jnp.float32),
)


def kernel(state, species_table, item_table, ability_table, move_table, W, b):
    idx = state[:, :NIDX].astype(jnp.int32).T
    emb = _sc_gather(idx, species_table, item_table, ability_table,
                     move_table)
    W10 = jnp.concatenate([
        W[:3 * D].reshape(3, D, D),
        jnp.broadcast_to(W[3 * D:4 * D] * (1.0 / 3.0), (3, D, D)),
        jnp.broadcast_to(W[4 * D:5 * D] * 0.25, (4, D, D)),
    ])
    Wp = jnp.concatenate([jnp.zeros((NIDX, D), W.dtype), W[5 * D:]], axis=0)
    b2 = b.reshape(1, D)
    return _tc_matmul(emb, state, W10, Wp, b2)


# speedup vs baseline: 4.3874x; 1.4575x over previous
import functools

import jax
import jax.numpy as jnp
from jax import lax
from jax.experimental import pallas as pl
from jax.experimental.pallas import tpu as pltpu
from jax.experimental.pallas import tpu_sc as plsc

B = 16384
D = 64
NIDX = 10

_NC = 2
_NS = 16
_NW = _NC * _NS
_RPW = B // _NW
_C = 64
_NCH = _RPW // _C

_mesh = plsc.VectorSubcoreMesh(core_axis_name="c", subcore_axis_name="s",
                               num_cores=_NC, num_subcores=_NS)


def _sc_gather_body(idx_hbm, sp_hbm, it_hbm, ab_hbm, mv_hbm, out_hbm,
                    idx_v, gbuf, gsem, wsem):
    wid = lax.axis_index("s") * _NC + lax.axis_index("c")
    base = wid * _RPW
    tables = (sp_hbm, it_hbm, ab_hbm, ab_hbm, ab_hbm, ab_hbm,
              mv_hbm, mv_hbm, mv_hbm, mv_hbm)

    pltpu.sync_copy(idx_hbm.at[:, pl.ds(base, _RPW)], idx_v)

    def fire_gathers(ci, par):
        cps = []
        for j, t in enumerate(tables):
            cps.append(pltpu.async_copy(
                t.at[idx_v.at[j, pl.ds(ci * _C, _C)]],
                gbuf.at[par, j], gsem.at[par]))
        return cps

    def fire_writes(ci, par):
        r = pl.multiple_of(base + ci * _C, _C)
        return [pltpu.async_copy(
            gbuf.at[par, j],
            out_hbm.at[j // 2, pl.ds(r, _C), pl.ds((j % 2) * D, D)], wsem)
            for j in range(NIDX)]

    g_cps = {0: fire_gathers(0, 0)}
    w_cps = {}
    for ci in range(_NCH):
        par = ci % 2
        if ci + 1 < _NCH:
            nxt = (ci + 1) % 2
            if ci >= 1:
                for cp in w_cps.pop(ci - 1):
                    cp.wait()
            g_cps[ci + 1] = fire_gathers(ci + 1, nxt)
        for cp in g_cps.pop(ci):
            cp.wait()
        w_cps[ci] = fire_writes(ci, par)
    for cps in w_cps.values():
        for cp in cps:
            cp.wait()


_sc_gather = functools.partial(
    pl.kernel,
    out_type=jax.ShapeDtypeStruct((NIDX // 2, B, 2 * D), jnp.float32),
    mesh=_mesh,
    compiler_params=pltpu.CompilerParams(use_tc_tiling_on_sc=False),
    scratch_types=[
        pltpu.VMEM((NIDX, _RPW), jnp.int32),
        pltpu.VMEM((2, NIDX, _C, D), jnp.float32),
        pltpu.SemaphoreType.DMA((2,)),
        pltpu.SemaphoreType.DMA,
    ],
)(_sc_gather_body)


_BC = 2048


def _tc_body(emb_ref, st_ref, wpair_ref, wp_ref, b_ref, out_ref):
    dn = (((1,), (0,)), ((), ()))
    acc = lax.dot_general(st_ref[...], wp_ref[...], dn,
                          preferred_element_type=jnp.float32)
    for p in range(NIDX // 2):
        acc = acc + lax.dot_general(emb_ref[p], wpair_ref[p], dn,
                                    preferred_element_type=jnp.float32)
    out_ref[...] = jnp.maximum(acc + b_ref[...], 0.0)


_tc_matmul = pl.pallas_call(
    _tc_body,
    grid=(B // _BC,),
    in_specs=[
        pl.BlockSpec((NIDX // 2, _BC, 2 * D), lambda i: (0, i, 0)),
        pl.BlockSpec((_BC, NIDX + 128), lambda i: (i, 0)),
        pl.BlockSpec((NIDX // 2, 2 * D, D), lambda i: (0, 0, 0)),
        pl.BlockSpec((NIDX + 128, D), lambda i: (0, 0)),
        pl.BlockSpec((1, D), lambda i: (0, 0)),
    ],
    out_specs=pl.BlockSpec((_BC, D), lambda i: (i, 0)),
    out_shape=jax.ShapeDtypeStruct((B, D), jnp.float32),
)


def kernel(state, species_table, item_table, ability_table, move_table, W, b):
    idx = state[:, :NIDX].astype(jnp.int32).T
    emb = _sc_gather(idx, species_table, item_table, ability_table,
                     move_table)
    W10 = jnp.concatenate([
        W[:3 * D].reshape(3, D, D),
        jnp.broadcast_to(W[3 * D:4 * D] * (1.0 / 3.0), (3, D, D)),
        jnp.broadcast_to(W[4 * D:5 * D] * 0.25, (4, D, D)),
    ])
    Wpair = W10.reshape(NIDX // 2, 2 * D, D)
    Wp = jnp.concatenate([jnp.zeros((NIDX, D), W.dtype), W[5 * D:]], axis=0)
    b2 = b.reshape(1, D)
    return _tc_matmul(emb, state, Wpair, Wp, b2)


# speedup vs baseline: 4.8708x; 1.1102x over previous
import functools

import jax
import jax.numpy as jnp
from jax import lax
from jax.experimental import pallas as pl
from jax.experimental.pallas import tpu as pltpu
from jax.experimental.pallas import tpu_sc as plsc

B = 16384
D = 64
NIDX = 10

_NC = 2
_NS = 16
_NW = _NC * _NS
_RPW = B // _NW
_C = 64
_NCH = _RPW // _C

_mesh = plsc.VectorSubcoreMesh(core_axis_name="c", subcore_axis_name="s",
                               num_cores=_NC, num_subcores=_NS)


def _sc_gather_body(idx_hbm, sp_hbm, it_hbm, ab_hbm, mv_hbm, out_hbm,
                    idx_v, gbuf, gsem, wsem):
    wid = lax.axis_index("s") * _NC + lax.axis_index("c")
    base = wid * _RPW
    tables = (sp_hbm, it_hbm, ab_hbm, ab_hbm, ab_hbm, ab_hbm,
              mv_hbm, mv_hbm, mv_hbm, mv_hbm)

    pltpu.sync_copy(idx_hbm.at[:, pl.ds(base, _RPW)], idx_v)

    def fire_gathers(ci, par):
        cps = []
        for j, t in enumerate(tables):
            cps.append(pltpu.async_copy(
                t.at[idx_v.at[j, pl.ds(ci * _C, _C)]],
                gbuf.at[par, j], gsem.at[par]))
        return cps

    def fire_writes(ci, par):
        r = pl.multiple_of(base + ci * _C, _C)
        return [pltpu.async_copy(
            gbuf.at[par, j],
            out_hbm.at[j // 2, pl.ds(r, _C), pl.ds((j % 2) * D, D)], wsem)
            for j in range(NIDX)]

    g_cps = {0: fire_gathers(0, 0)}
    w_cps = {}
    for ci in range(_NCH):
        par = ci % 2
        if ci + 1 < _NCH:
            nxt = (ci + 1) % 2
            if ci >= 1:
                for cp in w_cps.pop(ci - 1):
                    cp.wait()
            g_cps[ci + 1] = fire_gathers(ci + 1, nxt)
        for cp in g_cps.pop(ci):
            cp.wait()
        w_cps[ci] = fire_writes(ci, par)
    for cps in w_cps.values():
        for cp in cps:
            cp.wait()


_sc_gather = functools.partial(
    pl.kernel,
    out_type=jax.ShapeDtypeStruct((NIDX // 2, B, 2 * D), jnp.float32),
    mesh=_mesh,
    compiler_params=pltpu.CompilerParams(use_tc_tiling_on_sc=False),
    scratch_types=[
        pltpu.VMEM((NIDX, _RPW), jnp.int32),
        pltpu.VMEM((2, NIDX, _C, D), jnp.float32),
        pltpu.SemaphoreType.DMA((2,)),
        pltpu.SemaphoreType.DMA,
    ],
)(_sc_gather_body)


_BC = 2048


def _tc_body(emb_ref, stt_ref, wpairt_ref, wpt_ref, b_ref, out_ref):
    acc = lax.dot_general(wpt_ref[...], stt_ref[...], (((1,), (0,)), ((), ())),
                          preferred_element_type=jnp.float32)
    for p in range(NIDX // 2):
        acc = acc + lax.dot_general(wpairt_ref[p], emb_ref[p],
                                    (((1,), (1,)), ((), ())),
                                    preferred_element_type=jnp.float32)
    out_ref[...] = jnp.maximum(acc + b_ref[...], 0.0)


_tc_matmul = pl.pallas_call(
    _tc_body,
    grid=(B // _BC,),
    in_specs=[
        pl.BlockSpec((NIDX // 2, _BC, 2 * D), lambda i: (0, i, 0)),
        pl.BlockSpec((NIDX + 128, _BC), lambda i: (0, i)),
        pl.BlockSpec((NIDX // 2, D, 2 * D), lambda i: (0, 0, 0)),
        pl.BlockSpec((D, NIDX + 128), lambda i: (0, 0)),
        pl.BlockSpec((D, 1), lambda i: (0, 0)),
    ],
    out_specs=pl.BlockSpec((D, _BC), lambda i: (0, i)),
    out_shape=jax.ShapeDtypeStruct((D, B), jnp.float32),
)


def kernel(state, species_table, item_table, ability_table, move_table, W, b):
    idx = state[:, :NIDX].astype(jnp.int32).T
    emb = _sc_gather(idx, species_table, item_table, ability_table,
                     move_table)
    WT = W.T
    w3 = WT[:, 3 * D:4 * D] * (1.0 / 3.0)
    w4 = WT[:, 4 * D:5 * D] * 0.25
    WpairT = jnp.stack([
        WT[:, 0:2 * D],
        jnp.concatenate([WT[:, 2 * D:3 * D], w3], axis=1),
        jnp.concatenate([w3, w3], axis=1),
        jnp.concatenate([w4, w4], axis=1),
        jnp.concatenate([w4, w4], axis=1),
    ])
    WpT = jnp.concatenate([jnp.zeros((D, NIDX), W.dtype), WT[:, 5 * D:]],
                          axis=1)
    outT = _tc_matmul(emb, state.T, WpairT, WpT, b.reshape(D, 1))
    return outT.T


# speedup vs baseline: 5.1888x; 1.0653x over previous
import functools

import jax
import jax.numpy as jnp
from jax import lax
from jax.experimental import pallas as pl
from jax.experimental.pallas import tpu as pltpu
from jax.experimental.pallas import tpu_sc as plsc

B = 16384
D = 64
NIDX = 10

_NC = 2
_NS = 16
_NW = _NC * _NS
_RPW = B // _NW

_mesh = plsc.VectorSubcoreMesh(core_axis_name="c", subcore_axis_name="s",
                               num_cores=_NC, num_subcores=_NS)


def _build_sc_gather(num_tables, slot_table, chunk):
    nslots = len(slot_table)
    nch = _RPW // chunk

    def body(idx_hbm, *args):
        tabs = args[:num_tables]
        out_hbm = args[num_tables]
        idx_v, gbuf, gsem, wsem = args[num_tables + 1:]
        wid = lax.axis_index("s") * _NC + lax.axis_index("c")
        base = wid * _RPW
        tables = tuple(tabs[k] for k in slot_table)

        pltpu.sync_copy(idx_hbm.at[:, pl.ds(base, _RPW)], idx_v)

        def fire_gathers(ci, par):
            return [pltpu.async_copy(
                t.at[idx_v.at[j, pl.ds(ci * chunk, chunk)]],
                gbuf.at[par, j], gsem.at[par]) for j, t in enumerate(tables)]

        def fire_writes(ci, par):
            r = pl.multiple_of(base + ci * chunk, chunk)
            return [pltpu.async_copy(
                gbuf.at[par, j],
                out_hbm.at[j // 2, pl.ds(r, chunk), pl.ds((j % 2) * D, D)],
                wsem) for j in range(nslots)]

        g_cps = {0: fire_gathers(0, 0)}
        w_cps = {}
        for ci in range(nch):
            if ci + 1 < nch:
                if ci >= 1:
                    for cp in w_cps.pop(ci - 1):
                        cp.wait()
                g_cps[ci + 1] = fire_gathers(ci + 1, (ci + 1) % 2)
            for cp in g_cps.pop(ci):
                cp.wait()
            w_cps[ci] = fire_writes(ci, ci % 2)
        for cps in w_cps.values():
            for cp in cps:
                cp.wait()

    return functools.partial(
        pl.kernel,
        out_type=jax.ShapeDtypeStruct((nslots // 2, B, 2 * D), jnp.float32),
        mesh=_mesh,
        compiler_params=pltpu.CompilerParams(use_tc_tiling_on_sc=False),
        scratch_types=[
            pltpu.VMEM((nslots, _RPW), jnp.int32),
            pltpu.VMEM((2, nslots, chunk, D), jnp.float32),
            pltpu.SemaphoreType.DMA((2,)),
            pltpu.SemaphoreType.DMA,
        ],
    )(body)


_sc_small = _build_sc_gather(3, (0, 1, 1, 1, 1, 2, 2, 2), chunk=64)
_sc_species = _build_sc_gather(2, (0, 1), chunk=128)


_BC = 2048


def _tc_body(emb1_ref, emb2_ref, stt_ref, wp1_ref, wp2_ref, wpt_ref, b_ref,
             out_ref):
    acc = lax.dot_general(wpt_ref[...], stt_ref[...], (((1,), (0,)), ((), ())),
                          preferred_element_type=jnp.float32)
    dnT = (((1,), (1,)), ((), ()))
    for p in range(4):
        acc = acc + lax.dot_general(wp1_ref[p], emb1_ref[p], dnT,
                                    preferred_element_type=jnp.float32)
    acc = acc + lax.dot_general(wp2_ref[0], emb2_ref[0], dnT,
                                preferred_element_type=jnp.float32)
    out_ref[...] = jnp.maximum(acc + b_ref[...], 0.0)


_tc_matmul = pl.pallas_call(
    _tc_body,
    grid=(B // _BC,),
    in_specs=[
        pl.BlockSpec((4, _BC, 2 * D), lambda i: (0, i, 0)),
        pl.BlockSpec((1, _BC, 2 * D), lambda i: (0, i, 0)),
        pl.BlockSpec((NIDX + 128, _BC), lambda i: (0, i)),
        pl.BlockSpec((4, D, 2 * D), lambda i: (0, 0, 0)),
        pl.BlockSpec((1, D, 2 * D), lambda i: (0, 0, 0)),
        pl.BlockSpec((D, NIDX + 128), lambda i: (0, 0)),
        pl.BlockSpec((D, 1), lambda i: (0, 0)),
    ],
    out_specs=pl.BlockSpec((D, _BC), lambda i: (0, i)),
    out_shape=jax.ShapeDtypeStruct((D, B), jnp.float32),
)


def kernel(state, species_table, item_table, ability_table, move_table, W, b):
    idx_small = state[:, 1:9].astype(jnp.int32).T
    idx_sp = jnp.stack([state[:, 0], state[:, 9]]).astype(jnp.int32)
    emb1 = _sc_small(idx_small, item_table, ability_table, move_table)
    emb2 = _sc_species(idx_sp, species_table, move_table)
    WT = W.T
    w3 = WT[:, 3 * D:4 * D] * (1.0 / 3.0)
    w4 = WT[:, 4 * D:5 * D] * 0.25
    Wp1 = jnp.stack([
        WT[:, D:3 * D],
        jnp.concatenate([w3, w3], axis=1),
        jnp.concatenate([w3, w4], axis=1),
        jnp.concatenate([w4, w4], axis=1),
    ])
    Wp2 = jnp.concatenate([WT[:, 0:D], w4], axis=1)[None]
    WpT = jnp.concatenate([jnp.zeros((D, NIDX), W.dtype), WT[:, 5 * D:]],
                          axis=1)
    outT = _tc_matmul(emb1, emb2, state.T, Wp1, Wp2, WpT, b.reshape(D, 1))
    return outT.T
